# Initial kernel scaffold; baseline (speedup 1.0000x reference)
import jax
import jax.numpy as jnp
from jax.experimental import pallas as pl


def kernel(X, table, pe):
    raise NotImplementedError("write your pallas kernel here")


# speedup vs baseline: 2.8861x; 2.8861x over previous
import functools
import math

import jax
import jax.numpy as jnp
from jax import lax
from jax.experimental import pallas as pl
from jax.experimental.pallas import tpu as pltpu
from jax.experimental.pallas import tpu_sc as plsc

VOCAB = 100000
D = 64
SEQ = 200
NC = 2
NS = 16
NW = NC * NS
CH = 128
SCALE = math.sqrt(D)


def _sc_body(x_hbm, pe_hbm, table_hbm, out_hbm, idx_v, pe_v, in_v, ou_v, sg, sw):
    n_rows = x_hbm.shape[0] * x_hbm.shape[1] // NW
    n_ch = n_rows // CH
    wid = lax.axis_index("s") * NC + lax.axis_index("c")
    row_base = wid * n_rows

    pltpu.sync_copy(x_hbm.at[pl.ds(wid * n_ch, n_ch)], idx_v)
    pltpu.sync_copy(pe_hbm, pe_v)

    def compute(c, buf):
        p0 = lax.rem(c * CH, SEQ)
        def group_body(g, _):
            idx16 = idx_v[c, pl.ds(g * 16, 16)]
            scale16 = jnp.where(idx16 == 0, 0.0, SCALE)
            for jj in range(16):
                j = g * 16 + jj
                sv = jnp.full((16,), scale16[jj], jnp.float32)
                for q in range(D // 16):
                    row = in_v[buf, j, pl.ds(16 * q, 16)]
                    pev = pe_v[p0 + j, pl.ds(16 * q, 16)]
                    ou_v[buf, j, pl.ds(16 * q, 16)] = row * sv + pev
            return 0
        lax.fori_loop(0, CH // 16, group_body, 0)

    def start_gather(c, buf):
        pltpu.async_copy(table_hbm.at[idx_v.at[c]], in_v.at[buf], sg.at[buf])

    def wait_gather(buf):
        pltpu.make_async_copy(table_hbm.at[pl.ds(0, CH)], in_v.at[buf],
                              sg.at[buf]).wait()

    def start_write(c, buf):
        pltpu.async_copy(ou_v.at[buf],
                         out_hbm.at[pl.ds(row_base + c * CH, CH)], sw.at[buf])

    def wait_write(buf):
        pltpu.make_async_copy(table_hbm.at[pl.ds(0, CH)], ou_v.at[buf],
                              sw.at[buf]).wait()

    start_gather(0, 0)
    start_gather(1, 1)

    def pair_body(g, _):
        for b in range(2):
            c = 2 * g + b
            wait_gather(b)
            @pl.when(g >= 1)
            def _():
                wait_write(b)
            compute(c, b)
            @pl.when(g < n_ch // 2 - 1)
            def _():
                start_gather(c + 2, b)
            start_write(c, b)
        return 0

    lax.fori_loop(0, n_ch // 2, pair_body, 0)
    wait_write(0)
    wait_write(1)


@functools.partial(jax.jit, donate_argnums=())
def kernel(X, table, pe):
    B, T = X.shape
    n_rows = B * T // NW
    n_ch = n_rows // CH
    x2d = X.reshape(B * T // CH, CH)
    pe2 = jnp.concatenate([pe[:SEQ], pe[:SEQ]], axis=0)

    mesh = plsc.VectorSubcoreMesh(core_axis_name="c", subcore_axis_name="s",
                                  num_cores=NC, num_subcores=NS)
    out = pl.kernel(
        _sc_body,
        out_type=jax.ShapeDtypeStruct((B * T, D), jnp.float32),
        mesh=mesh,
        compiler_params=pltpu.CompilerParams(use_tc_tiling_on_sc=False),
        scratch_types=[
            pltpu.VMEM((n_ch, CH), jnp.int32),
            pltpu.VMEM((2 * SEQ, D), jnp.float32),
            pltpu.VMEM((2, CH, D), jnp.float32),
            pltpu.VMEM((2, CH, D), jnp.float32),
            pltpu.SemaphoreType.DMA((2,)),
            pltpu.SemaphoreType.DMA((2,)),
        ],
    )(x2d, pe2, table)
    return out.reshape(B, T, D)


# speedup vs baseline: 4.0932x; 1.4182x over previous
import functools
import math

import jax
import jax.numpy as jnp
from jax import lax
from jax.experimental import pallas as pl
from jax.experimental.pallas import tpu as pltpu
from jax.experimental.pallas import tpu_sc as plsc

VOCAB = 100000
D = 64
SEQ = 200
NC = 2
NS = 16
NW = NC * NS
CH = 128
SCALE = math.sqrt(D)


def _sc_body(x_hbm, pe_hbm, table_hbm, out_hbm, idx_v, pe_v, in_v, ou_v, sg, sw):
    n_rows = x_hbm.shape[0] * x_hbm.shape[1] // NW
    n_ch = n_rows // CH
    wid = lax.axis_index("s") * NC + lax.axis_index("c")
    row_base = wid * n_rows

    pltpu.sync_copy(x_hbm.at[pl.ds(wid * n_ch, n_ch)], idx_v)
    pltpu.sync_copy(pe_hbm, pe_v)

    def compute(c, buf):
        p0 = lax.rem(c * CH, SEQ)
        def group_body(g, _):
            idx16 = idx_v[c, pl.ds(g * 16, 16)]
            scale16 = jnp.where(idx16 == 0, 0.0, SCALE)
            for jj in range(16):
                j = g * 16 + jj
                sv = jnp.full((16,), scale16[jj], jnp.float32)
                for q in range(D // 16):
                    row = in_v[buf, j, pl.ds(16 * q, 16)]
                    pev = pe_v[p0 + j, pl.ds(16 * q, 16)]
                    ou_v[buf, j, pl.ds(16 * q, 16)] = row * sv + pev
            return 0
        lax.fori_loop(0, CH // 16, group_body, 0)

    def start_gather(c, buf):
        pltpu.async_copy(table_hbm.at[idx_v.at[c]], in_v.at[buf], sg.at[buf])

    def wait_gather(buf):
        pltpu.make_async_copy(table_hbm.at[pl.ds(0, CH)], in_v.at[buf],
                              sg.at[buf]).wait()

    def start_write(c, buf):
        pltpu.async_copy(ou_v.at[buf],
                         out_hbm.at[pl.ds(row_base + c * CH, CH), pl.ds(0, D)],
                         sw.at[buf])

    def wait_write(buf):
        pltpu.make_async_copy(table_hbm.at[pl.ds(0, CH)], ou_v.at[buf],
                              sw.at[buf]).wait()

    start_gather(0, 0)
    start_gather(1, 1)

    def pair_body(g, _):
        for b in range(2):
            c = 2 * g + b
            wait_gather(b)
            @pl.when(g >= 1)
            def _():
                wait_write(b)
            compute(c, b)
            @pl.when(g < n_ch // 2 - 1)
            def _():
                start_gather(c + 2, b)
            start_write(c, b)
        return 0

    lax.fori_loop(0, n_ch // 2, pair_body, 0)
    wait_write(0)
    wait_write(1)


@functools.partial(jax.jit, donate_argnums=())
def kernel(X, table, pe):
    B, T = X.shape
    n_rows = B * T // NW
    n_ch = n_rows // CH
    x2d = X.reshape(B * T // CH, CH)
    pe2 = jnp.concatenate([pe[:SEQ], pe[:SEQ]], axis=0)

    mesh = plsc.VectorSubcoreMesh(core_axis_name="c", subcore_axis_name="s",
                                  num_cores=NC, num_subcores=NS)
    out = pl.kernel(
        _sc_body,
        out_type=jax.ShapeDtypeStruct((B * T, 128), jnp.float32),
        mesh=mesh,
        compiler_params=pltpu.CompilerParams(use_tc_tiling_on_sc=False),
        scratch_types=[
            pltpu.VMEM((n_ch, CH), jnp.int32),
            pltpu.VMEM((2 * SEQ, D), jnp.float32),
            pltpu.VMEM((2, CH, D), jnp.float32),
            pltpu.VMEM((2, CH, D), jnp.float32),
            pltpu.SemaphoreType.DMA((2,)),
            pltpu.SemaphoreType.DMA((2,)),
        ],
    )(x2d, pe2, table)
    return out[:, :D].reshape(B, T, D)


# speedup vs baseline: 7.0580x; 1.7243x over previous
import functools
import math

import jax
import jax.numpy as jnp
from jax import lax
from jax.experimental import pallas as pl
from jax.experimental.pallas import tpu as pltpu
from jax.experimental.pallas import tpu_sc as plsc

D = 64
NC = 2
NS = 16
NW = NC * NS
CH = 128
SCALE = math.sqrt(D)


def _sc_body(x_hbm, pe_hbm, table_hbm, out_hbm, idx_v, pe_v, in_v, ou_v, sg, sw):
    seq = x_hbm.shape[0]
    wid = lax.axis_index("s") * NC + lax.axis_index("c")

    pltpu.sync_copy(x_hbm.at[:, wid], idx_v)
    pltpu.sync_copy(pe_hbm, pe_v)

    def compute(c, buf):
        pev = [pe_v[c, pl.ds(16 * q, 16)] for q in range(D // 16)]
        def group_body(g, _):
            idx16 = idx_v[c, pl.ds(g * 16, 16)]
            scale16 = jnp.where(idx16 == 0, 0.0, SCALE)
            for jj in range(16):
                j = g * 16 + jj
                sv = jnp.full((16,), scale16[jj], jnp.float32)
                for q in range(D // 16):
                    row = in_v[buf, j, pl.ds(16 * q, 16)]
                    ou_v[buf, j, pl.ds(16 * q, 16)] = row * sv + pev[q]
            return 0
        lax.fori_loop(0, CH // 16, group_body, 0)

    def start_gather(c, buf):
        pltpu.async_copy(table_hbm.at[idx_v.at[c]], in_v.at[buf], sg.at[buf])

    def wait_gather(buf):
        pltpu.make_async_copy(table_hbm.at[pl.ds(0, CH)], in_v.at[buf],
                              sg.at[buf]).wait()

    def start_write(c, buf):
        pltpu.async_copy(ou_v.at[buf],
                         out_hbm.at[pl.ds(wid * CH, CH), c, pl.ds(0, D)],
                         sw.at[buf])

    def wait_write(buf):
        pltpu.make_async_copy(table_hbm.at[pl.ds(0, CH)], ou_v.at[buf],
                              sw.at[buf]).wait()

    start_gather(0, 0)
    start_gather(1, 1)

    def pair_body(g, _):
        for b in range(2):
            c = 2 * g + b
            wait_gather(b)
            @pl.when(g >= 1)
            def _():
                wait_write(b)
            compute(c, b)
            @pl.when(g < seq // 2 - 1)
            def _():
                start_gather(c + 2, b)
            start_write(c, b)
        return 0

    lax.fori_loop(0, seq // 2, pair_body, 0)
    wait_write(0)
    wait_write(1)


@jax.jit
def kernel(X, table, pe):
    B, T = X.shape
    x3d = X.T.reshape(T, NW, CH)
    pe2 = pe[:T]

    mesh = plsc.VectorSubcoreMesh(core_axis_name="c", subcore_axis_name="s",
                                  num_cores=NC, num_subcores=NS)
    out = pl.kernel(
        _sc_body,
        out_type=jax.ShapeDtypeStruct((B, T, 128), jnp.float32),
        mesh=mesh,
        compiler_params=pltpu.CompilerParams(use_tc_tiling_on_sc=False),
        scratch_types=[
            pltpu.VMEM((T, CH), jnp.int32),
            pltpu.VMEM((T, D), jnp.float32),
            pltpu.VMEM((2, CH, D), jnp.float32),
            pltpu.VMEM((2, CH, D), jnp.float32),
            pltpu.SemaphoreType.DMA((2,)),
            pltpu.SemaphoreType.DMA((2,)),
        ],
    )(x3d, pe2, table)
    return out[:, :, :D].reshape(B, T, D)


# speedup vs baseline: 7.5147x; 1.0647x over previous
import functools
import math

import jax
import jax.numpy as jnp
from jax import lax
from jax.experimental import pallas as pl
from jax.experimental.pallas import tpu as pltpu
from jax.experimental.pallas import tpu_sc as plsc

D = 64
NC = 2
NS = 16
NW = NC * NS
CH = 128
NBUF = 4
SCALE = math.sqrt(D)


def _sc_body(x_hbm, pe_hbm, table_hbm, out_hbm, idx_v, pe_v, in_v, ou_v, sg, sw):
    seq = x_hbm.shape[0]
    wid = lax.axis_index("s") * NC + lax.axis_index("c")

    pltpu.sync_copy(x_hbm.at[:, wid], idx_v)
    pltpu.sync_copy(pe_hbm, pe_v)

    def compute(c, buf):
        pev = [pe_v[c, pl.ds(16 * q, 16)] for q in range(D // 16)]
        def group_body(g, _):
            idx16 = idx_v[c, pl.ds(g * 16, 16)]
            scale16 = jnp.where(idx16 == 0, 0.0, SCALE)
            for jj in range(16):
                j = g * 16 + jj
                sv = jnp.full((16,), scale16[jj], jnp.float32)
                for q in range(D // 16):
                    row = in_v[buf, j, pl.ds(16 * q, 16)]
                    ou_v[buf, j, pl.ds(16 * q, 16)] = row * sv + pev[q]
            return 0
        lax.fori_loop(0, CH // 16, group_body, 0)

    def start_gather(c, buf):
        pltpu.async_copy(table_hbm.at[idx_v.at[c]], in_v.at[buf], sg.at[buf])

    def wait_gather(buf):
        pltpu.make_async_copy(table_hbm.at[pl.ds(0, CH)], in_v.at[buf],
                              sg.at[buf]).wait()

    def start_write(c, buf):
        pltpu.async_copy(ou_v.at[buf],
                         out_hbm.at[pl.ds(wid * CH, CH), c, pl.ds(0, D)],
                         sw.at[buf])

    def wait_write(buf):
        pltpu.make_async_copy(table_hbm.at[pl.ds(0, CH)], ou_v.at[buf],
                              sw.at[buf]).wait()

    for b in range(NBUF):
        start_gather(b, b)

    def ring_body(g, _):
        for b in range(NBUF):
            c = NBUF * g + b
            wait_gather(b)
            @pl.when(g >= 1)
            def _():
                wait_write(b)
            compute(c, b)
            @pl.when(g < seq // NBUF - 1)
            def _():
                start_gather(c + NBUF, b)
            start_write(c, b)
        return 0

    lax.fori_loop(0, seq // NBUF, ring_body, 0)
    for b in range(NBUF):
        wait_write(b)


@jax.jit
def kernel(X, table, pe):
    B, T = X.shape
    x3d = X.T.reshape(T, NW, CH)
    pe2 = pe[:T]

    mesh = plsc.VectorSubcoreMesh(core_axis_name="c", subcore_axis_name="s",
                                  num_cores=NC, num_subcores=NS)
    out = pl.kernel(
        _sc_body,
        out_type=jax.ShapeDtypeStruct((B, T, 128), jnp.float32),
        mesh=mesh,
        compiler_params=pltpu.CompilerParams(use_tc_tiling_on_sc=False),
        scratch_types=[
            pltpu.VMEM((T, CH), jnp.int32),
            pltpu.VMEM((T, D), jnp.float32),
            pltpu.VMEM((NBUF, CH, D), jnp.float32),
            pltpu.VMEM((NBUF, CH, D), jnp.float32),
            pltpu.SemaphoreType.DMA((NBUF,)),
            pltpu.SemaphoreType.DMA((NBUF,)),
        ],
    )(x3d, pe2, table)
    return out[:, :, :D].reshape(B, T, D)
